# Initial kernel scaffold; baseline (speedup 1.0000x reference)
import jax
import jax.numpy as jnp
from jax.experimental import pallas as pl


def kernel(table, labels):
    raise NotImplementedError("write your pallas kernel here")


# speedup vs baseline: 3.1102x; 3.1102x over previous
import functools

import jax
import jax.numpy as jnp
from jax import lax
from jax.experimental import pallas as pl

CAP = 64
K = 64
REG = 0.1
ROWS = 4096
D = 512
BLK = 512


def _scores_body(t_ref, labr_ref, labc_ref, sc_ref, scl_ref, *, b_scale):
    t = t_ref[...]
    labr = labr_ref[...]
    labc = labc_ref[...]
    onehot = (labr == lax.broadcasted_iota(jnp.int32, (K, ROWS), 0)
              ).astype(jnp.float32)
    counts = jnp.sum(onehot, axis=1, keepdims=True)
    sums = jnp.dot(onehot, t, preferred_element_type=jnp.float32)
    centers = sums / jnp.clip(counts, 1.0)
    onehot_t = (labc == lax.broadcasted_iota(jnp.int32, (ROWS, K), 1)
                ).astype(jnp.float32)
    cdata = jnp.dot(onehot_t, centers,
                    preferred_element_type=jnp.float32)
    diff = t - cdata
    dist = jnp.sum(diff * diff, axis=1, keepdims=True) * (1.0 / D)
    mn = jnp.min(dist)
    mx = jnp.max(dist)
    dn = (dist - mn) / (mx - mn)
    scores = dn + labc.astype(jnp.float32)
    smn = jnp.min(scores)
    smx = jnp.max(scores)
    sc_ref[...] = scores
    scl_ref[...] = (scores - smn) / (smx - smn) * b_scale


def _rank_body(sclc_ref, sclr_ref, scc_ref, scr_ref, r_ref, ri_ref):
    i = pl.program_id(0)
    si = sclc_ref[...]
    sr = sclr_ref[...]
    x = (si - sr) * (1.0 / REG)
    sig = 1.0 / (1.0 + jnp.exp(-x))
    r_ref[...] = 0.5 + jnp.sum(sig, axis=1, keepdims=True)
    vi = scc_ref[...]
    vr = scr_ref[...]
    jglob = lax.broadcasted_iota(jnp.int32, (BLK, ROWS), 1)
    iglob = i * BLK + lax.broadcasted_iota(jnp.int32, (BLK, ROWS), 0)
    before = (vr < vi) | ((vr == vi) & (jglob < iglob))
    cnt = jnp.sum(before.astype(jnp.int32), axis=1, keepdims=True)
    ri_ref[...] = cnt // CAP + 1


def kernel(table, labels):
    rows = table.shape[1]
    d = table.shape[-1]
    table = table.reshape(-1, rows, d)
    b = table.shape[0]
    t2d = table.reshape(rows, d)
    labr = labels.reshape(1, rows).astype(jnp.int32)
    labc = labels.reshape(rows, 1).astype(jnp.int32)

    scores, scaled = pl.pallas_call(
        functools.partial(_scores_body, b_scale=float(b)),
        out_shape=(
            jax.ShapeDtypeStruct((rows, 1), jnp.float32),
            jax.ShapeDtypeStruct((rows, 1), jnp.float32),
        ),
    )(t2d, labr, labc)

    sclr = scaled.reshape(1, rows)
    scr = scores.reshape(1, rows)
    grid = rows // BLK
    ranks, rank_idx = pl.pallas_call(
        _rank_body,
        grid=(grid,),
        in_specs=[
            pl.BlockSpec((BLK, 1), lambda i: (i, 0)),
            pl.BlockSpec((1, rows), lambda i: (0, 0)),
            pl.BlockSpec((BLK, 1), lambda i: (i, 0)),
            pl.BlockSpec((1, rows), lambda i: (0, 0)),
        ],
        out_specs=(
            pl.BlockSpec((BLK, 1), lambda i: (i, 0)),
            pl.BlockSpec((BLK, 1), lambda i: (i, 0)),
        ),
        out_shape=(
            jax.ShapeDtypeStruct((rows, 1), jnp.float32),
            jax.ShapeDtypeStruct((rows, 1), jnp.int32),
        ),
    )(scaled, sclr, scores, scr)

    return (ranks.reshape(b, rows, 1),
            rank_idx.reshape(b, rows, 1),
            scores.reshape(b, rows, 1))
